# Initial kernel scaffold; baseline (speedup 1.0000x reference)
import jax
import jax.numpy as jnp
from jax.experimental import pallas as pl


def kernel(coords0, coords1, desc0, desc1, params):
    raise NotImplementedError("write your pallas kernel here")


# speedup vs baseline: 1.8554x; 1.8554x over previous
import math

import numpy as np
import jax
import jax.numpy as jnp
from jax.experimental import pallas as pl
from jax.experimental.pallas import tpu as pltpu

FD = 128
NHD = 4
HD = FD // NHD
KN = 9
NPT = 2048
CHUNK = 256
NCH = NPT // CHUNK
EPS = 1e-5


def _stats(arrs, cnt):
    s = arrs[0].sum(axis=0, keepdims=True)
    for a in arrs[1:]:
        s = s + a.sum(axis=0, keepdims=True)
    m = s / cnt
    v = ((arrs[0] - m) ** 2).sum(axis=0, keepdims=True)
    for a in arrs[1:]:
        v = v + ((a - m) ** 2).sum(axis=0, keepdims=True)
    v = v / cnt
    return m, jax.lax.rsqrt(v + EPS)


def _split3(f):
    f0 = f.astype(jnp.bfloat16)
    r = f - f0.astype(jnp.float32)
    f1 = r.astype(jnp.bfloat16)
    f2 = (r - f1.astype(jnp.float32)).astype(jnp.bfloat16)
    return f0, f1, f2


def _gat(ohb, parts):
    g = jnp.dot(ohb, parts[0], preferred_element_type=jnp.float32)
    g = g + jnp.dot(ohb, parts[1], preferred_element_type=jnp.float32)
    g = g + jnp.dot(ohb, parts[2], preferred_element_type=jnp.float32)
    return g


def _relu(x):
    return jnp.maximum(x, 0.0)


def _leaky(x):
    return jnp.where(x >= 0, x, 0.2 * x)


def _self1_kernel(feats_ref, pts_ref, idx_ref, wa_ref, wb_ref, w1t_ref,
                  wga_ref, wgb_ref, x1_ref, fang_ref, x1o_ref,
                  ya0, ya1, ya2, yg0, yg1, yg2, vm_ref):
    fsp = _split3(feats_ref[0])
    psp = _split3(pts_ref[0])
    lane = jax.lax.broadcasted_iota(jnp.int32, (CHUNK, NPT), 1)
    yas = (ya0, ya1, ya2)
    ygs = (yg0, yg1, yg2)
    wga = wga_ref[...]

    def chunk_body(c, carry):
        s1, q1 = carry
        r0 = c * CHUNK
        idx_c = idx_ref[0, pl.ds(r0, CHUNK), :]
        fe = feats_ref[0, pl.ds(r0, CHUNK), :]
        pc = pts_ref[0, pl.ds(r0, CHUNK), :]
        acc = [jnp.zeros((CHUNK, FD), jnp.float32) for _ in range(3)]
        accg = [jnp.zeros((CHUNK, FD), jnp.float32) for _ in range(3)]
        vmax = jnp.full((CHUNK, FD), -jnp.inf, jnp.float32)
        for k in range(KN):
            j, t = k // 3, k % 3
            oh = (lane == idx_c[:, k:k + 1]).astype(jnp.bfloat16)
            nf = _gat(oh, fsp)
            nc = _gat(oh, psp)
            xk = jnp.concatenate([fe, nf - fe], axis=1)
            acc[j] = acc[j] + jnp.dot(xk, wa_ref[t],
                                      preferred_element_type=jnp.float32)
            v = jnp.dot(xk, w1t_ref[...], preferred_element_type=jnp.float32)
            vmax = jnp.maximum(vmax, v)
            s1 = s1 + jnp.sum(v, axis=0, keepdims=True)
            q1 = q1 + jnp.sum(v * v, axis=0, keepdims=True)
            dot = jnp.sum(pc * nc, axis=1, keepdims=True)
            den = jnp.sqrt(jnp.sum(pc * pc, axis=1, keepdims=True)) * \
                jnp.sqrt(jnp.sum(nc * nc, axis=1, keepdims=True))
            cos = dot / den
            accg[j] = accg[j] + cos * wga[t:t + 1, :]
        for j in range(3):
            yas[j][pl.ds(r0, CHUNK), :] = acc[j]
            ygs[j][pl.ds(r0, CHUNK), :] = accg[j]
        vm_ref[pl.ds(r0, CHUNK), :] = vmax
        return s1, q1

    s1, q1 = jax.lax.fori_loop(
        0, NCH, chunk_body,
        (jnp.zeros((1, FD), jnp.float32), jnp.zeros((1, FD), jnp.float32)))

    ya = [ya0[...], ya1[...], ya2[...]]
    m, r = _stats(ya, 3.0 * NPT)
    x1a = [_relu((a - m) * r) for a in ya]
    yb = jnp.dot(x1a[0], wb_ref[0], preferred_element_type=jnp.float32)
    for j in (1, 2):
        yb = yb + jnp.dot(x1a[j], wb_ref[j], preferred_element_type=jnp.float32)
    mb, rb = _stats([yb], float(NPT))
    x1_ref[0] = _relu((yb - mb) * rb)

    yg = [yg0[...], yg1[...], yg2[...]]
    mg, rg = _stats(yg, 3.0 * NPT)
    g1 = [_relu((a - mg) * rg) for a in yg]
    gb = jnp.dot(g1[0], wgb_ref[0], preferred_element_type=jnp.float32)
    for j in (1, 2):
        gb = gb + jnp.dot(g1[j], wgb_ref[j], preferred_element_type=jnp.float32)
    mgb, rgb = _stats([gb], float(NPT))
    fang_ref[0] = _relu((gb - mgb) * rgb)

    cnt = 9.0 * NPT
    mv = s1 / cnt
    rv = jax.lax.rsqrt(q1 / cnt - mv * mv + EPS)
    x1o_ref[0] = _leaky((vm_ref[...] - mv) * rv)


def _self2_kernel(x0_ref, x1_ref, x1o_ref, fang_ref, idx_ref, wa2_ref,
                  wb2_ref, w2t_ref, w3t_ref, w3ot_ref, out_ref,
                  ya0, ya1, ya2, vm_ref):
    x1 = x1_ref[0]
    x1o = x1o_ref[0]
    x1sp = _split3(x1)
    x1osp = _split3(x1o)
    lane = jax.lax.broadcasted_iota(jnp.int32, (CHUNK, NPT), 1)
    yas = (ya0, ya1, ya2)

    def chunk_body(c, carry):
        s2, q2 = carry
        r0 = c * CHUNK
        idx_c = idx_ref[0, pl.ds(r0, CHUNK), :]
        f1 = x1_ref[0, pl.ds(r0, CHUNK), :]
        f1o = x1o_ref[0, pl.ds(r0, CHUNK), :]
        acc = [jnp.zeros((CHUNK, FD), jnp.float32) for _ in range(3)]
        vmax = jnp.full((CHUNK, 2 * FD), -jnp.inf, jnp.float32)
        for k in range(KN):
            j, t = k // 3, k % 3
            oh = (lane == idx_c[:, k:k + 1]).astype(jnp.bfloat16)
            nf1 = _gat(oh, x1sp)
            nf1o = _gat(oh, x1osp)
            xk = jnp.concatenate([f1, nf1 - f1], axis=1)
            xko = jnp.concatenate([f1o, nf1o - f1o], axis=1)
            acc[j] = acc[j] + jnp.dot(xk, wa2_ref[t],
                                      preferred_element_type=jnp.float32)
            v = jnp.dot(xko, w2t_ref[...], preferred_element_type=jnp.float32)
            vmax = jnp.maximum(vmax, v)
            s2 = s2 + jnp.sum(v, axis=0, keepdims=True)
            q2 = q2 + jnp.sum(v * v, axis=0, keepdims=True)
        for j in range(3):
            yas[j][pl.ds(r0, CHUNK), :] = acc[j]
        vm_ref[pl.ds(r0, CHUNK), :] = vmax
        return s2, q2

    s2, q2 = jax.lax.fori_loop(
        0, NCH, chunk_body,
        (jnp.zeros((1, 2 * FD), jnp.float32),
         jnp.zeros((1, 2 * FD), jnp.float32)))

    ya = [ya0[...], ya1[...], ya2[...]]
    m, r = _stats(ya, 3.0 * NPT)
    x2a = [_relu((a - m) * r) for a in ya]
    yb = jnp.dot(x2a[0], wb2_ref[0], preferred_element_type=jnp.float32)
    for j in (1, 2):
        yb = yb + jnp.dot(x2a[j], wb2_ref[j], preferred_element_type=jnp.float32)
    mb, rb = _stats([yb], float(NPT))
    x2 = _relu((yb - mb) * rb)

    cnt = 9.0 * NPT
    mv = s2 / cnt
    rv = jax.lax.rsqrt(q2 / cnt - mv * mv + EPS)
    x2o = _leaky((vm_ref[...] - mv) * rv)

    x0 = x0_ref[0]
    fang = fang_ref[0]
    w3t = w3t_ref[...]
    a1 = x1 + fang
    a2 = x2 + fang
    z3 = jnp.dot(x0, w3t[:FD], preferred_element_type=jnp.float32)
    z3 = z3 + jnp.dot(a1, w3t[FD:2 * FD], preferred_element_type=jnp.float32)
    z3 = z3 + jnp.dot(a2, w3t[2 * FD:], preferred_element_type=jnp.float32)
    m3, r3 = _stats([z3], float(NPT))
    o3 = _leaky((z3 - m3) * r3)

    w3ot = w3ot_ref[...]
    z3o = jnp.dot(x0, w3ot[:FD], preferred_element_type=jnp.float32)
    z3o = z3o + jnp.dot(x1o, w3ot[FD:2 * FD],
                        preferred_element_type=jnp.float32)
    z3o = z3o + jnp.dot(x2o, w3ot[2 * FD:],
                        preferred_element_type=jnp.float32)
    m3o, r3o = _stats([z3o], float(NPT))
    out_ref[0] = o3 + _leaky((z3o - m3o) * r3o)


def _qkv_kernel(x_ref, w_ref, b_ref, o_ref):
    o_ref[0] = jnp.dot(x_ref[0], w_ref[...],
                       preferred_element_type=jnp.float32) + b_ref[...]


def _attn_kernel(q_ref, kv_ref, x_ref, wm_ref, bm_ref, w1_ref, y_ref, ps_ref):
    t = pl.program_id(1)
    q = q_ref[0]
    kv = kv_ref[0]
    scale = 1.0 / math.sqrt(HD)
    outs = []
    for h in range(NHD):
        qh = q[:, h * HD:(h + 1) * HD]
        kh = kv[:, FD + h * HD:FD + (h + 1) * HD]
        vh = kv[:, 2 * FD + h * HD:2 * FD + (h + 1) * HD]
        s = jax.lax.dot_general(qh, kh, (((1,), (1,)), ((), ())),
                                preferred_element_type=jnp.float32) * scale
        s = s - jnp.max(s, axis=1, keepdims=True)
        p = jnp.exp(s)
        p = p / jnp.sum(p, axis=1, keepdims=True)
        outs.append(jnp.dot(p, vh, preferred_element_type=jnp.float32))
    att = jnp.concatenate(outs, axis=1)
    msg = jnp.dot(att, wm_ref[...],
                  preferred_element_type=jnp.float32) + bm_ref[...]
    x = x_ref[0]
    w1 = w1_ref[...]
    y = jnp.dot(x, w1[:FD], preferred_element_type=jnp.float32) + \
        jnp.dot(msg, w1[FD:], preferred_element_type=jnp.float32)
    y_ref[0] = y

    @pl.when(t == 0)
    def _():
        ps_ref[0] = jnp.zeros((2, 2 * FD), jnp.float32)

    ps_ref[0, 0:1, :] += jnp.sum(y, axis=0, keepdims=True)
    ps_ref[0, 1:2, :] += jnp.sum(y * y, axis=0, keepdims=True)


def _final_kernel(y_ref, st_ref, x_ref, w2_ref, b2_ref, o_ref):
    m = st_ref[0, 0:1, :]
    r = st_ref[0, 1:2, :]
    h = _relu((y_ref[0] - m) * r)
    o_ref[0] = x_ref[0] + jnp.dot(h, w2_ref[...],
                                  preferred_element_type=jnp.float32) + b2_ref[...]


def _w3(w):
    return jnp.transpose(w[:, :, 0, :], (2, 1, 0))


def kernel(coords0, coords1, desc0, desc1, params):
    p = params
    f32 = jnp.float32
    pts = jnp.concatenate([jnp.swapaxes(coords0, 1, 2),
                           jnp.swapaxes(coords1, 1, 2)], axis=0)
    feats = jnp.concatenate([jnp.swapaxes(desc0, 1, 2),
                             jnp.swapaxes(desc1, 1, 2)], axis=0)

    wa1 = _w3(p['sa_ac1a_w'])
    wb1 = _w3(p['sa_ac1b_w'])
    wga = jnp.transpose(p['sa_anga_w'][:, 0, 0, :])
    wgb = _w3(p['sa_angb_w'])
    wa2 = _w3(p['sa_ac2a_w'])
    wb2 = _w3(p['sa_ac2b_w'])
    w1t = jnp.transpose(p['sa_conv1_w'])
    w2t = jnp.transpose(p['sa_conv2_w'])
    w3t = jnp.transpose(p['sa_conv3_w'])
    w3ot = jnp.transpose(p['sa_conv3o_w'])

    perm = np.arange(FD).reshape(HD, NHD).T.reshape(FD)
    wqkv = jnp.concatenate([jnp.transpose(p['ap_q_w'])[:, perm],
                            jnp.transpose(p['ap_k_w'])[:, perm],
                            jnp.transpose(p['ap_v_w'])[:, perm]], axis=1)
    bqkv = jnp.concatenate([p['ap_q_b'][perm], p['ap_k_b'][perm],
                            p['ap_v_b'][perm]]).reshape(1, 3 * FD)
    wm = jnp.transpose(p['ap_m_w'])[perm, :]
    bm = p['ap_m_b'].reshape(1, FD)
    wmlp1t = jnp.transpose(p['ap_mlp1_w'])
    wmlp2t = jnp.transpose(p['ap_mlp2_w'])
    bmlp2 = p['ap_mlp2_b'].reshape(1, FD)

    whole = lambda a: pl.BlockSpec(a.shape, lambda *ix: (0,) * a.ndim)

    def _knn(c):
        q = jnp.swapaxes(c, 1, 2)
        d = -2.0 * jnp.matmul(q, jnp.swapaxes(q, 1, 2))
        n2 = jnp.sum(q ** 2, axis=-1)
        d = jnp.clip(d + n2[:, :, None] + n2[:, None, :], 1e-12, None)
        _, idx = jax.lax.top_k(-d, KN + 1)
        return idx[:, :, 1:]

    idx9 = jnp.concatenate([_knn(coords0), _knn(coords1)], axis=0)

    sN = lambda last: pl.BlockSpec((1, NPT, last), lambda s: (s, 0, 0))
    vmem = pltpu.VMEM
    x1, fang, x1o = pl.pallas_call(
        _self1_kernel,
        grid=(4,),
        in_specs=[sN(FD), sN(2), sN(KN), whole(wa1), whole(wb1), whole(w1t),
                  whole(wga), whole(wgb)],
        out_specs=[sN(FD), sN(FD), sN(FD)],
        out_shape=[jax.ShapeDtypeStruct((4, NPT, FD), f32)] * 3,
        scratch_shapes=[vmem((NPT, FD), f32)] * 7,
    )(feats, pts, idx9, wa1, wb1, w1t, wga, wgb)

    sa = pl.pallas_call(
        _self2_kernel,
        grid=(4,),
        in_specs=[sN(FD), sN(FD), sN(FD), sN(FD), sN(KN), whole(wa2),
                  whole(wb2), whole(w2t), whole(w3t), whole(w3ot)],
        out_specs=sN(FD),
        out_shape=jax.ShapeDtypeStruct((4, NPT, FD), f32),
        scratch_shapes=[vmem((NPT, FD), f32)] * 3 + [vmem((NPT, 2 * FD), f32)],
    )(feats, x1, x1o, fang, idx9, wa2, wb2, w2t, w3t, w3ot)

    qkv = pl.pallas_call(
        _qkv_kernel,
        grid=(4, NCH),
        in_specs=[pl.BlockSpec((1, CHUNK, FD), lambda s, t: (s, t, 0)),
                  whole(wqkv), whole(bqkv)],
        out_specs=pl.BlockSpec((1, CHUNK, 3 * FD), lambda s, t: (s, t, 0)),
        out_shape=jax.ShapeDtypeStruct((4, NPT, 3 * FD), f32),
    )(sa, wqkv, bqkv)

    y, ps = pl.pallas_call(
        _attn_kernel,
        grid=(4, NCH),
        in_specs=[pl.BlockSpec((1, CHUNK, 3 * FD), lambda s, t: (s, t, 0)),
                  pl.BlockSpec((1, NPT, 3 * FD), lambda s, t: ((s + 2) % 4, 0, 0)),
                  pl.BlockSpec((1, CHUNK, FD), lambda s, t: (s, t, 0)),
                  whole(wm), whole(bm), whole(wmlp1t)],
        out_specs=[pl.BlockSpec((1, CHUNK, 2 * FD), lambda s, t: (s, t, 0)),
                   pl.BlockSpec((1, 2, 2 * FD), lambda s, t: (s, 0, 0))],
        out_shape=[jax.ShapeDtypeStruct((4, NPT, 2 * FD), f32),
                   jax.ShapeDtypeStruct((4, 2, 2 * FD), f32)],
    )(qkv, qkv, sa, wm, bm, wmlp1t)

    mean = ps[:, 0:1, :] / NPT
    rstd = jax.lax.rsqrt(ps[:, 1:2, :] / NPT - mean * mean + EPS)
    st = jnp.concatenate([mean, rstd], axis=1)

    out = pl.pallas_call(
        _final_kernel,
        grid=(4, NCH),
        in_specs=[pl.BlockSpec((1, CHUNK, 2 * FD), lambda s, t: (s, t, 0)),
                  pl.BlockSpec((1, 2, 2 * FD), lambda s, t: (s, 0, 0)),
                  pl.BlockSpec((1, CHUNK, FD), lambda s, t: (s, t, 0)),
                  whole(wmlp2t), whole(bmlp2)],
        out_specs=pl.BlockSpec((1, CHUNK, FD), lambda s, t: (s, t, 0)),
        out_shape=jax.ShapeDtypeStruct((4, NPT, FD), f32),
    )(y, st, sa, wmlp2t, bmlp2)

    out_t = jnp.swapaxes(out, 1, 2)
    return out_t[0:2], out_t[2:4]
